# baseline (speedup 1.0000x reference)
import functools

import jax
import jax.numpy as jnp
from jax import lax
from jax.experimental import pallas as pl
from jax.experimental.pallas import tpu as pltpu
from jax.experimental.pallas import tpu_sc as plsc

NUM_EMB = 100000
DIM = 64
BATCH = 4096 * 50

NUM_CORES = 2
NUM_SUBCORES = 16
NW = NUM_CORES * NUM_SUBCORES
B_PER_W = BATCH // NW
CHUNK = 400
NCHUNK = B_PER_W // CHUNK
NBUF = 4

_mesh = plsc.VectorSubcoreMesh(core_axis_name="c", subcore_axis_name="s")


@functools.partial(
    pl.kernel,
    mesh=_mesh,
    compiler_params=pltpu.CompilerParams(use_tc_tiling_on_sc=False),
    out_type=jax.ShapeDtypeStruct((BATCH, DIM), jnp.bfloat16),
    scratch_types=[
        pltpu.VMEM((B_PER_W,), jnp.int32),
        [pltpu.VMEM((CHUNK, DIM), jnp.bfloat16) for _ in range(NBUF)],
        [pltpu.SemaphoreType.DMA for _ in range(NBUF)],
        [pltpu.SemaphoreType.DMA for _ in range(NBUF)],
    ],
)
def _emb_lookup(ids_hbm, table_hbm, out_hbm, idx_v, bufs, gsems, wsems):
    wid = lax.axis_index("s") * NUM_CORES + lax.axis_index("c")
    base = wid * B_PER_W
    pltpu.sync_copy(ids_hbm.at[pl.ds(base, B_PER_W)], idx_v)

    ghandles = [None] * NBUF
    whandles = [None] * NBUF

    def start_gather(j):
        b = j % NBUF
        if whandles[b] is not None:
            whandles[b].wait()
        ghandles[b] = pltpu.async_copy(
            table_hbm.at[idx_v.at[pl.ds(j * CHUNK, CHUNK)]], bufs[b], gsems[b])

    for j in range(min(NBUF - 1, NCHUNK)):
        start_gather(j)

    for i in range(NCHUNK):
        b = i % NBUF
        j = i + NBUF - 1
        if j < NCHUNK:
            start_gather(j)
        ghandles[b].wait()
        whandles[b] = pltpu.async_copy(
            bufs[b], out_hbm.at[pl.ds(base + i * CHUNK, CHUNK)], wsems[b])
    for b in range(NBUF):
        if whandles[b] is not None:
            whandles[b].wait()


def kernel(token_ids, weight):
    wbf = weight.astype(jnp.bfloat16)
    table2 = jnp.pad(wbf, ((0, 0), (0, 128 - DIM))).reshape(2 * NUM_EMB, DIM)
    flat_ids = token_ids.reshape(-1).astype(jnp.int32) * 2
    out = _emb_lookup(flat_ids, table2)
    return out.astype(jnp.float32).reshape(token_ids.shape + (DIM,))


# speedup vs baseline: 1.8953x; 1.8953x over previous
import functools

import jax
import jax.numpy as jnp
from jax import lax
from jax.experimental import pallas as pl
from jax.experimental.pallas import tpu as pltpu
from jax.experimental.pallas import tpu_sc as plsc

NUM_EMB = 100000
DIM = 64
BATCH = 4096 * 50

NUM_CORES = 2
NUM_SUBCORES = 16
NW = NUM_CORES * NUM_SUBCORES
B_PER_W = BATCH // NW
CHUNK = 400
NCHUNK = B_PER_W // CHUNK
NBUF = 4


@functools.cache
def _build_emb_lookup():
    mesh = plsc.VectorSubcoreMesh(
        core_axis_name="c", subcore_axis_name="s",
        num_cores=NUM_CORES, num_subcores=NUM_SUBCORES)

    @functools.partial(
        pl.kernel,
        mesh=mesh,
        compiler_params=pltpu.CompilerParams(use_tc_tiling_on_sc=False),
        out_type=jax.ShapeDtypeStruct((BATCH, DIM), jnp.float32),
        scratch_types=[
            pltpu.VMEM((B_PER_W,), jnp.int32),
            [pltpu.VMEM((CHUNK, DIM), jnp.float32) for _ in range(NBUF)],
            [pltpu.SemaphoreType.DMA for _ in range(NBUF)],
            [pltpu.SemaphoreType.DMA for _ in range(NBUF)],
        ],
    )
    def _emb_lookup(ids_hbm, table_hbm, out_hbm, idx_v, bufs, gsems, wsems):
        wid = lax.axis_index("s") * NUM_CORES + lax.axis_index("c")
        base = wid * B_PER_W
        pltpu.sync_copy(ids_hbm.at[pl.ds(base, B_PER_W)], idx_v)

        ghandles = [None] * NBUF
        whandles = [None] * NBUF

        def start_gather(j):
            b = j % NBUF
            if whandles[b] is not None:
                whandles[b].wait()
            ghandles[b] = pltpu.async_copy(
                table_hbm.at[idx_v.at[pl.ds(j * CHUNK, CHUNK)]],
                bufs[b], gsems[b])

        for j in range(min(NBUF - 1, NCHUNK)):
            start_gather(j)

        for i in range(NCHUNK):
            b = i % NBUF
            j = i + NBUF - 1
            if j < NCHUNK:
                start_gather(j)
            ghandles[b].wait()
            whandles[b] = pltpu.async_copy(
                bufs[b], out_hbm.at[pl.ds(base + i * CHUNK, CHUNK)], wsems[b])
        for b in range(NBUF):
            if whandles[b] is not None:
                whandles[b].wait()

    return _emb_lookup


def kernel(token_ids, weight):
    table2 = jnp.pad(weight, ((0, 0), (0, 128 - DIM))).reshape(2 * NUM_EMB, DIM)
    flat_ids = token_ids.reshape(-1).astype(jnp.int32) * 2
    out = _build_emb_lookup()(flat_ids, table2)
    return out.reshape(token_ids.shape + (DIM,))
